# baseline (speedup 1.0000x reference)
import functools

import jax
import jax.numpy as jnp
from jax import lax
from jax.experimental import pallas as pl
from jax.experimental.pallas import tpu as pltpu
from jax.experimental.pallas import tpu_sc as plsc

D = 128
L = 16
CHUNK = 128
NBUF = 2


def _make_kernel(B, NC, NS):
    NW = NC * NS
    b_per_w = B // NW
    n_chunks = b_per_w // CHUNK
    n_groups = CHUNK // L
    mesh = plsc.VectorSubcoreMesh(core_axis_name="c", subcore_axis_name="s")

    @functools.partial(
        pl.kernel,
        mesh=mesh,
        out_type=[
            jax.ShapeDtypeStruct((B,), jnp.float32),
            jax.ShapeDtypeStruct((B,), jnp.float32),
        ],
        scratch_types=[
            pltpu.VMEM((b_per_w,), jnp.int32),
            pltpu.VMEM((b_per_w,), jnp.int32),
            pltpu.VMEM((b_per_w,), jnp.int32),
            pltpu.VMEM((NBUF, CHUNK, D), jnp.float32),
            pltpu.VMEM((NBUF, CHUNK, D), jnp.float32),
            pltpu.VMEM((NBUF, CHUNK, D), jnp.float32),
            pltpu.VMEM((b_per_w,), jnp.float32),
            pltpu.VMEM((b_per_w,), jnp.float32),
        ] + [pltpu.SemaphoreType.DMA] * NBUF,
    )
    def k(users_h, pos_h, neg_h, ue_h, ie_h, out_p_h, out_n_h,
          uidx, pidx, nidx, urows, prows, nrows, outp, outn, *sems):
        sem0 = sems[0]
        wid = lax.axis_index("s") * NC + lax.axis_index("c")
        base = wid * b_per_w

        pltpu.async_copy(users_h.at[pl.ds(base, b_per_w)], uidx, sem0)
        pltpu.async_copy(pos_h.at[pl.ds(base, b_per_w)], pidx, sem0)
        pltpu.async_copy(neg_h.at[pl.ds(base, b_per_w)], nidx, sem0)
        pltpu.make_async_copy(users_h.at[pl.ds(base, b_per_w)], uidx, sem0).wait()
        pltpu.make_async_copy(pos_h.at[pl.ds(base, b_per_w)], pidx, sem0).wait()
        pltpu.make_async_copy(neg_h.at[pl.ds(base, b_per_w)], nidx, sem0).wait()

        lanes = lax.iota(jnp.int32, L)

        def fire(j):
            s = j % NBUF
            sl = pl.ds(j * CHUNK, CHUNK)
            pltpu.async_copy(ue_h.at[uidx.at[sl]], urows.at[s], sems[s])
            pltpu.async_copy(ie_h.at[pidx.at[sl]], prows.at[s], sems[s])
            pltpu.async_copy(ie_h.at[nidx.at[sl]], nrows.at[s], sems[s])

        def drain(j):
            s = j % NBUF
            sl = pl.ds(j * CHUNK, CHUNK)
            pltpu.make_async_copy(ue_h.at[uidx.at[sl]], urows.at[s], sems[s]).wait()
            pltpu.make_async_copy(ie_h.at[pidx.at[sl]], prows.at[s], sems[s]).wait()
            pltpu.make_async_copy(ie_h.at[nidx.at[sl]], nrows.at[s], sems[s]).wait()

        for j in range(min(NBUF, n_chunks)):
            fire(j)

        for j in range(n_chunks):
            with jax.named_scope(f"drain{j}"):
                drain(j)
            s = j % NBUF

            def group_body(g, carry, j=j, s=s):
                totp = jnp.zeros((L,), jnp.float32)
                totn = jnp.zeros((L,), jnp.float32)
                for rr in range(L):
                    r = g * L + rr
                    ap = jnp.zeros((L,), jnp.float32)
                    an = jnp.zeros((L,), jnp.float32)
                    for kk in range(D // L):
                        uvec = urows[s, r, pl.ds(kk * L, L)]
                        ap = ap + uvec * prows[s, r, pl.ds(kk * L, L)]
                        an = an + uvec * nrows[s, r, pl.ds(kk * L, L)]
                    for sh in (8, 4, 2, 1):
                        perm = jnp.bitwise_xor(lanes, sh)
                        ap = ap + ap.at[perm].get(mode="promise_in_bounds")
                        an = an + an.at[perm].get(mode="promise_in_bounds")
                    sel = lanes == rr
                    totp = jnp.where(sel, ap, totp)
                    totn = jnp.where(sel, an, totn)
                outp[pl.ds(j * CHUNK + g * L, L)] = totp
                outn[pl.ds(j * CHUNK + g * L, L)] = totn
                return carry

            with jax.named_scope(f"compute{j}"):
                lax.fori_loop(0, n_groups, group_body, 0)
            if j + NBUF < n_chunks:
                fire(j + NBUF)

        pltpu.sync_copy(outp, out_p_h.at[pl.ds(base, b_per_w)])
        pltpu.sync_copy(outn, out_n_h.at[pl.ds(base, b_per_w)])

    return k


def kernel(users, positive_items, negative_items, user_embeddings, item_embeddings):
    B = users.shape[0]
    info = plsc.get_sparse_core_info()
    k = _make_kernel(B, info.num_cores, info.num_subcores)
    out_p, out_n = k(
        users.astype(jnp.int32),
        positive_items.astype(jnp.int32),
        negative_items.astype(jnp.int32),
        user_embeddings,
        item_embeddings,
    )
    return out_p, out_n


# speedup vs baseline: 1.0144x; 1.0144x over previous
import functools

import jax
import jax.numpy as jnp
from jax import lax
from jax.experimental import pallas as pl
from jax.experimental.pallas import tpu as pltpu
from jax.experimental.pallas import tpu_sc as plsc

D = 128
L = 16
CMAX = 128
NBUF = 2


def _chunk_schedule(total):
    sizes = []
    for c in (32, 96):
        if sum(sizes) + c <= total:
            sizes.append(c)
    while sum(sizes) < total:
        sizes.append(min(CMAX, total - sum(sizes)))
    return sizes


def _make_kernel(B, NC, NS):
    NW = NC * NS
    b_per_w = B // NW
    sizes = _chunk_schedule(b_per_w)
    offs = [sum(sizes[:i]) for i in range(len(sizes))]
    n_chunks = len(sizes)
    mesh = plsc.VectorSubcoreMesh(core_axis_name="c", subcore_axis_name="s")

    row_buf = lambda: pltpu.VMEM((CMAX, D), jnp.float32)

    @functools.partial(
        pl.kernel,
        mesh=mesh,
        out_type=[
            jax.ShapeDtypeStruct((B,), jnp.float32),
            jax.ShapeDtypeStruct((B,), jnp.float32),
        ],
        scratch_types=[
            pltpu.VMEM((b_per_w,), jnp.int32),
            pltpu.VMEM((b_per_w,), jnp.int32),
            pltpu.VMEM((b_per_w,), jnp.int32),
            row_buf(), row_buf(),
            row_buf(), row_buf(),
            row_buf(), row_buf(),
            pltpu.VMEM((b_per_w,), jnp.float32),
            pltpu.VMEM((b_per_w,), jnp.float32),
            pltpu.SemaphoreType.DMA,
            pltpu.SemaphoreType.DMA,
            pltpu.SemaphoreType.DMA,
        ],
    )
    def k(users_h, pos_h, neg_h, ue_h, ie_h, out_p_h, out_n_h,
          uidx, pidx, nidx, u0, u1, p0, p1, n0, n1, outp, outn,
          sem_idx, sem_a, sem_b):
        ubufs, pbufs, nbufs = (u0, u1), (p0, p1), (n0, n1)
        sems = (sem_a, sem_b)
        wid = lax.axis_index("s") * NC + lax.axis_index("c")
        base = wid * b_per_w

        idx_src = (users_h, pos_h, neg_h)
        idx_dst = (uidx, pidx, nidx)
        for src, dst in zip(idx_src, idx_dst):
            pltpu.async_copy(src.at[pl.ds(base, b_per_w)], dst, sem_idx)
        for src, dst in zip(idx_src, idx_dst):
            pltpu.make_async_copy(src.at[pl.ds(base, b_per_w)], dst, sem_idx).wait()

        lanes = lax.iota(jnp.int32, L)
        perms = {d: jnp.bitwise_xor(lanes, d) for d in (1, 2, 4, 8)}
        masks = {d: (lanes & d) == 0 for d in (1, 2, 4, 8)}

        def plan(j):
            s = j % NBUF
            c = sizes[j]
            sl = pl.ds(offs[j], c)
            return (
                (ue_h.at[uidx.at[sl]], ubufs[s].at[pl.ds(0, c)], sems[s]),
                (ie_h.at[pidx.at[sl]], pbufs[s].at[pl.ds(0, c)], sems[s]),
                (ie_h.at[nidx.at[sl]], nbufs[s].at[pl.ds(0, c)], sems[s]),
            )

        def fire(j):
            for src, dst, sem in plan(j):
                pltpu.async_copy(src, dst, sem)

        def drain(j):
            for src, dst, sem in plan(j):
                pltpu.make_async_copy(src, dst, sem).wait()

        def treesum(vecs):
            d = 1
            while len(vecs) > 1:
                pd, md, nxt = perms[d], masks[d], []
                for a, b in zip(vecs[0::2], vecs[1::2]):
                    pa = a.at[pd].get(mode="promise_in_bounds")
                    pb = b.at[pd].get(mode="promise_in_bounds")
                    nxt.append(jnp.where(md, a, pb) + jnp.where(md, pa, b))
                vecs, d = nxt, d * 2
            return vecs[0]

        for j in range(min(NBUF, n_chunks)):
            fire(j)

        for j in range(n_chunks):
            drain(j)
            s = j % NBUF
            ub, pb, nb = ubufs[s], pbufs[s], nbufs[s]

            def group_body(g, carry, j=j, ub=ub, pb=pb, nb=nb):
                vp, vn = [], []
                for rr in range(L):
                    r = g * L + rr
                    ap = jnp.zeros((L,), jnp.float32)
                    an = jnp.zeros((L,), jnp.float32)
                    for kk in range(D // L):
                        uvec = ub[r, pl.ds(kk * L, L)]
                        ap = ap + uvec * pb[r, pl.ds(kk * L, L)]
                        an = an + uvec * nb[r, pl.ds(kk * L, L)]
                    vp.append(ap)
                    vn.append(an)
                outp[pl.ds(offs[j] + g * L, L)] = treesum(vp)
                outn[pl.ds(offs[j] + g * L, L)] = treesum(vn)
                return carry

            lax.fori_loop(0, sizes[j] // L, group_body, 0)

            if j + NBUF < n_chunks:
                fire(j + NBUF)

        pltpu.sync_copy(outp, out_p_h.at[pl.ds(base, b_per_w)])
        pltpu.sync_copy(outn, out_n_h.at[pl.ds(base, b_per_w)])

    return k


def kernel(users, positive_items, negative_items, user_embeddings, item_embeddings):
    B = users.shape[0]
    info = plsc.get_sparse_core_info()
    k = _make_kernel(B, info.num_cores, info.num_subcores)
    out_p, out_n = k(
        users.astype(jnp.int32),
        positive_items.astype(jnp.int32),
        negative_items.astype(jnp.int32),
        user_embeddings,
        item_embeddings,
    )
    return out_p, out_n
